# baseline (speedup 1.0000x reference)
import functools

import jax
import jax.numpy as jnp
from jax.experimental import pallas as pl
from jax.experimental.pallas import tpu as pltpu
from jax.experimental.pallas import tpu_sc as plsc

CUTOFF = 5.0
MAXNB = 64
RB = 256
BN = 32
GWIN = 128

_HIGH = jax.lax.Precision.DEFAULT


def _silu(x):
    return x * jax.nn.sigmoid(x)


def _dot(a, b):
    return jax.lax.dot_general(a, b, (((1,), (0,)), ((), ())),
                               precision=_HIGH,
                               preferred_element_type=jnp.float32)


def _build_graph(pos, r, batch, max_nb):
    n = pos.shape[0]
    chunk = 500
    nchunks = n // chunk
    jcol = jnp.arange(n)

    def _chunk(args):
        pc, bc, base = args
        diff = pc[:, None, :] - pos[None, :, :]
        d2 = jnp.sum(diff * diff, axis=-1)
        mask = (d2 < r * r) & (bc[:, None] == batch[None, :])
        rows = jnp.arange(chunk)
        mask = mask.at[rows, rows + base].set(False)
        score = jnp.where(mask, -jcol, -n)
        vals = jax.lax.top_k(score, max_nb)[0]
        return (-vals).astype(jnp.int32), vals > -n

    pos_c = pos.reshape(nchunks, chunk, 3)
    batch_c = batch.reshape(nchunks, chunk)
    bases = jnp.arange(nchunks) * chunk
    src_c, valid_c = jax.lax.map(_chunk, (pos_c, batch_c, bases))
    return src_c.reshape(n * max_nb), valid_c.reshape(n * max_nb)


def _gather_rows(table, idx1d):
    out_rows = idx1d.shape[0]
    width = table.shape[1]
    nworkers = 32
    b_per_w = out_rows // nworkers
    nit = b_per_w // GWIN
    mesh = plsc.VectorSubcoreMesh(core_axis_name="c", subcore_axis_name="s")

    @pl.kernel(out_type=jax.ShapeDtypeStruct((out_rows, width), table.dtype),
               mesh=mesh,
               scratch_types=[pltpu.VMEM((b_per_w,), jnp.int32),
                              pltpu.VMEM((GWIN, width), table.dtype),
                              pltpu.VMEM((GWIN, width), table.dtype),
                              pltpu.SemaphoreType.DMA,
                              pltpu.SemaphoreType.DMA])
    def k(x_hbm, i_hbm, o_hbm, idx_v, rows0, rows1, sem0, sem1):
        wid = jax.lax.axis_index("s") * 2 + jax.lax.axis_index("c")
        base = wid * b_per_w
        pltpu.sync_copy(i_hbm.at[pl.ds(base, b_per_w)], idx_v)
        rows = (rows0, rows1)
        sems = (sem0, sem1)

        @pl.loop(0, nit, step=2)
        def _(i):
            cps = []
            for b in range(2):
                cps.append(pltpu.async_copy(
                    x_hbm.at[idx_v.at[pl.ds((i + b) * GWIN, GWIN)]],
                    rows[b], sems[b]))
            for b in range(2):
                cps[b].wait()
                pltpu.sync_copy(
                    rows[b], o_hbm.at[pl.ds(base + (i + b) * GWIN, GWIN)])

    return k(table, idx1d)


def _embed_kernel(at_ref, emb_ref, out_ref):
    at = at_ref[...]
    ioc = jax.lax.broadcasted_iota(jnp.int32, (RB, 32), 1)
    oh = (at == ioc).astype(jnp.float32)
    out_ref[...] = _dot(oh, emb_ref[...])


def _geometry_kernel(nrbf, ps_ref, pr_ref, vm_ref, geo_ref):
    ps = ps_ref[...]
    pr = pr_ref[...]
    vec = ps[:, 0:3] - pr[:, 0:3]
    d2 = jnp.sum(vec * vec, axis=1, keepdims=True)
    dist = jnp.sqrt(d2 + 1e-12)
    dirv = vec / dist
    step = CUTOFF / (nrbf - 1)
    centers = jax.lax.broadcasted_iota(
        jnp.int32, (1, 24), 1).astype(jnp.float32) * step
    rbf = jnp.exp(-10.0 * (dist - centers) ** 2)
    fcut = 0.5 * (jnp.cos(jnp.pi * jnp.clip(dist / CUTOFF, 0.0, 1.0)) + 1.0)
    vm = vm_ref[...]
    geo_ref[:, 0:nrbf] = (rbf * fcut * vm)[:, 0:nrbf]
    geo_ref[:, nrbf:nrbf + 1] = fcut * vm
    geo_ref[:, nrbf + 1:24] = dirv * vm


def _edge_kernel(nrbf, h, has_xv, *refs):
    if has_xv:
        geo_ref, ps_ref, xv_ref, wf_ref, dxs_ref, dxv_ref = refs
    else:
        geo_ref, ps_ref, wf_ref, dxs_ref, dxv_ref = refs
    geo = geo_ref[...]
    wr = _dot(geo, wf_ref[...])
    m = ps_ref[...] * wr
    ds = m[:, 0:h]
    dvv = m[:, h:2 * h]
    dvs = m[:, 2 * h:3 * h]
    dx = geo[:, nrbf + 1:nrbf + 2]
    dy = geo[:, nrbf + 2:nrbf + 3]
    dz = geo[:, nrbf + 3:nrbf + 4]
    if has_xv:
        xv = xv_ref[...]
        dvx = dvv * dx + dvs * xv[:, 0:h]
        dvy = dvv * dy + dvs * xv[:, h:2 * h]
        dvz = dvv * dz + dvs * xv[:, 2 * h:3 * h]
    else:
        dvx = dvv * dx
        dvy = dvv * dy
        dvz = dvv * dz

    def red(a):
        return jnp.sum(a.reshape(BN, MAXNB, h), axis=1)

    dxs_ref[...] = red(ds)
    dxv_ref[:, 0:h] = red(dvx)
    dxv_ref[:, h:2 * h] = red(dvy)
    dxv_ref[:, 2 * h:3 * h] = red(dvz)


def _phi_kernel(xs_ref, w1_ref, b1_ref, w2_ref, b2_ref, out_ref):
    hid = _silu(_dot(xs_ref[...], w1_ref[...]) + b1_ref[...])
    out_ref[...] = _dot(hid, w2_ref[...]) + b2_ref[...]


def _update_kernel(h, xs_ref, xv_ref, dxs_ref, dxv_ref, wu_ref, wv_ref,
                   w1a_ref, w1b_ref, b1_ref, w2_ref, b2_ref,
                   xso_ref, xvo_ref):
    xs1 = xs_ref[...] + dxs_ref[...]
    xv1 = xv_ref[...] + dxv_ref[...]
    wu = wu_ref[...]
    wv = wv_ref[...]
    xvx = xv1[:, 0:h]
    xvy = xv1[:, h:2 * h]
    xvz = xv1[:, 2 * h:3 * h]
    ux = _dot(xvx, wu)
    uy = _dot(xvy, wu)
    uz = _dot(xvz, wu)
    vx = _dot(xvx, wv)
    vy = _dot(xvy, wv)
    vz = _dot(xvz, wv)
    vn = jnp.sqrt(vx * vx + vy * vy + vz * vz + 1e-12)
    hid = _silu(_dot(xs1, w1a_ref[...]) + _dot(vn, w1b_ref[...]) + b1_ref[...])
    a = _dot(hid, w2_ref[...]) + b2_ref[...]
    a_ss = a[:, 0:h]
    a_sv = a[:, h:2 * h]
    a_vv = a[:, 2 * h:3 * h]
    uv = ux * vx + uy * vy + uz * vz
    xso_ref[...] = xs1 + a_ss + a_sv * uv
    xvo_ref[:, 0:h] = xvx + a_vv * ux
    xvo_ref[:, h:2 * h] = xvy + a_vv * uy
    xvo_ref[:, 2 * h:3 * h] = xvz + a_vv * uz


def _head_kernel(xs_ref, at_ref, ft_ref, w1_ref, b1_ref, w2_ref, b2_ref,
                 sp_ref, out_ref):
    pid = pl.program_id(0)
    hid = _silu(_dot(xs_ref[...], w1_ref[...]) + b1_ref[...])
    e = (_dot(hid, w2_ref[...]) + b2_ref[...]) * ft_ref[...]
    ioc = jax.lax.broadcasted_iota(jnp.int32, (RB, 32), 1)
    oh = (at_ref[...] == ioc).astype(jnp.float32)
    ae = _dot(oh, sp_ref[...])
    s = jnp.sum(e + ae)

    @pl.when(pid == 0)
    def _():
        out_ref[...] = jnp.zeros_like(out_ref)

    out_ref[...] += s


def kernel(at_no, pos, shifts, batch, at_filter, params):
    n = pos.shape[0]
    h = params["embed"].shape[1]
    z = params["atom_sp"].shape[0]
    nrbf = params["msg"][0]["Wf"].shape[0]
    nlayers = len(params["msg"])

    np_ = ((n + RB - 1) // RB) * RB
    e = n * MAXNB
    ep = np_ * MAXNB
    f32 = jnp.float32

    src, valid = _build_graph(pos, CUTOFF, batch, MAXNB)
    src2d = jnp.concatenate([src, jnp.zeros((ep - e,), jnp.int32)])
    valid_f = jnp.zeros((ep, 1), f32).at[:e, 0].set(valid.astype(f32))

    pos_t = jnp.zeros((np_, 128), f32).at[:n, 0:3].set(pos + shifts)
    pos_d = jnp.zeros((np_, 4), f32).at[:n, 0:3].set(pos)
    pos_rep = jnp.repeat(pos_d, MAXNB, axis=0)

    at_p = jnp.full((np_, 1), z, jnp.int32).at[:n, 0].set(at_no)
    ft_p = jnp.zeros((np_, 1), f32).at[:n, 0].set(at_filter.astype(f32))
    emb_p = jnp.zeros((32, h), f32).at[:z].set(params["embed"])
    sp_p = jnp.zeros((32, 1), f32).at[:z, 0].set(params["atom_sp"])

    ngrid = np_ // RB
    egrid = ep // (BN * MAXNB)
    be = BN * MAXNB

    def full(shape):
        return pl.BlockSpec(shape, lambda i: tuple(0 for _ in shape))

    def rows(shape):
        return pl.BlockSpec(shape, lambda i: (i,) + tuple(0 for _ in shape[1:]))

    xs = pl.pallas_call(
        _embed_kernel,
        grid=(ngrid,),
        in_specs=[rows((RB, 1)), full((32, h))],
        out_specs=rows((RB, h)),
        out_shape=jax.ShapeDtypeStruct((np_, h), f32),
    )(at_p, emb_p)

    ps_g = _gather_rows(pos_t, src2d)
    geo = pl.pallas_call(
        functools.partial(_geometry_kernel, nrbf),
        grid=(egrid,),
        in_specs=[rows((be, 128)), rows((be, 4)), rows((be, 1))],
        out_specs=rows((be, 24)),
        out_shape=jax.ShapeDtypeStruct((ep, 24), f32),
    )(ps_g, pos_rep, valid_f)

    xv = jnp.zeros((np_, 3 * h), f32)
    for l in range(nlayers):
        p = params["msg"][l]
        wfp = (jnp.zeros((24, 3 * h), f32)
               .at[:nrbf].set(p["Wf"]).at[nrbf].set(p["bf"]))
        phi = pl.pallas_call(
            _phi_kernel,
            grid=(ngrid,),
            in_specs=[rows((RB, h)), full((h, h)), full((1, h)),
                      full((h, 3 * h)), full((1, 3 * h))],
            out_specs=rows((RB, 3 * h)),
            out_shape=jax.ShapeDtypeStruct((np_, 3 * h), f32),
        )(xs, p["W1"], p["b1"].reshape(1, h), p["W2"],
          p["b2"].reshape(1, 3 * h))

        phi_src = _gather_rows(phi, src2d)
        if l == 0:
            ins = [geo, phi_src, wfp]
            specs = [rows((be, 24)), rows((be, 3 * h)), full((24, 3 * h))]
        else:
            xv_src = _gather_rows(xv, src2d)
            ins = [geo, phi_src, xv_src, wfp]
            specs = [rows((be, 24)), rows((be, 3 * h)), rows((be, 3 * h)),
                     full((24, 3 * h))]
        dxs, dxv = pl.pallas_call(
            functools.partial(_edge_kernel, nrbf, h, l > 0),
            grid=(egrid,),
            in_specs=specs,
            out_specs=[rows((BN, h)), rows((BN, 3 * h))],
            out_shape=[jax.ShapeDtypeStruct((np_, h), f32),
                       jax.ShapeDtypeStruct((np_, 3 * h), f32)],
        )(*ins)

        q = params["upd"][l]
        xs, xv = pl.pallas_call(
            functools.partial(_update_kernel, h),
            grid=(ngrid,),
            in_specs=[rows((RB, h)), rows((RB, 3 * h)), rows((RB, h)),
                      rows((RB, 3 * h)), full((h, h)), full((h, h)),
                      full((h, h)), full((h, h)), full((1, h)),
                      full((h, 3 * h)), full((1, 3 * h))],
            out_specs=[rows((RB, h)), rows((RB, 3 * h))],
            out_shape=[jax.ShapeDtypeStruct((np_, h), f32),
                       jax.ShapeDtypeStruct((np_, 3 * h), f32)],
        )(xs, xv, dxs, dxv, q["WU"], q["WV"], q["W1"][:h], q["W1"][h:],
          q["b1"].reshape(1, h), q["W2"], q["b2"].reshape(1, 3 * h))

    o = params["out"]
    h2 = o["W1"].shape[1]
    out = pl.pallas_call(
        _head_kernel,
        grid=(ngrid,),
        in_specs=[rows((RB, h)), rows((RB, 1)), rows((RB, 1)),
                  full((h, h2)), full((1, h2)), full((h2, 1)), full((1, 1)),
                  full((32, 1))],
        out_specs=full((1, 1)),
        out_shape=jax.ShapeDtypeStruct((1, 1), f32),
    )(xs, at_p, ft_p, o["W1"], o["b1"].reshape(1, h2), o["W2"],
      o["b2"].reshape(1, 1), sp_p)
    return out.reshape(1)


# speedup vs baseline: 1.6655x; 1.6655x over previous
import functools

import jax
import jax.numpy as jnp
from jax.experimental import pallas as pl
from jax.experimental.pallas import tpu as pltpu
from jax.experimental.pallas import tpu_sc as plsc

CUTOFF = 5.0
MAXNB = 64
RB = 256
BN = 32
GWIN = 128

_HIGH = jax.lax.Precision.DEFAULT


def _silu(x):
    return x * jax.nn.sigmoid(x)


def _dot(a, b):
    return jax.lax.dot_general(a, b, (((1,), (0,)), ((), ())),
                               precision=_HIGH,
                               preferred_element_type=jnp.float32)


def _build_graph(pos, r, batch, max_nb):
    n = pos.shape[0]
    chunk = 500
    nchunks = n // chunk
    jcol = jnp.arange(n)

    def _chunk(args):
        pc, bc, base = args
        diff = pc[:, None, :] - pos[None, :, :]
        d2 = jnp.sum(diff * diff, axis=-1)
        mask = (d2 < r * r) & (bc[:, None] == batch[None, :])
        rows = jnp.arange(chunk)
        mask = mask.at[rows, rows + base].set(False)
        score = jnp.where(mask, -jcol, -n)
        vals = jax.lax.top_k(score, max_nb)[0]
        return (-vals).astype(jnp.int32), vals > -n

    pos_c = pos.reshape(nchunks, chunk, 3)
    batch_c = batch.reshape(nchunks, chunk)
    bases = jnp.arange(nchunks) * chunk
    src_c, valid_c = jax.lax.map(_chunk, (pos_c, batch_c, bases))
    return src_c.reshape(n * max_nb), valid_c.reshape(n * max_nb)


def _gather_rows(table, idx1d):
    out_rows = idx1d.shape[0]
    width = table.shape[1]
    nworkers = 32
    b_per_w = out_rows // nworkers
    nit = b_per_w // GWIN
    mesh = plsc.VectorSubcoreMesh(core_axis_name="c", subcore_axis_name="s")

    idx2d = idx1d.reshape(1, out_rows)

    @pl.kernel(out_type=jax.ShapeDtypeStruct((out_rows, width), table.dtype),
               mesh=mesh)
    def k(x_hbm, i_hbm, o_hbm):
        wid = jax.lax.axis_index("s") * 2 + jax.lax.axis_index("c")
        base = wid * nit

        def body(i_vmem, o_vmem):
            pltpu.sync_copy(x_hbm.at[i_vmem.at[0]], o_vmem)

        pltpu.emit_pipeline(
            body,
            grid=(nit,),
            in_specs=[pl.BlockSpec((1, GWIN), lambda i: (0, base + i))],
            out_specs=[pl.BlockSpec((GWIN, width), lambda i: (base + i, 0))],
            dimension_semantics=(pltpu.ARBITRARY,),
        )(i_hbm, o_hbm)

    return k(table, idx2d)


def _embed_kernel(at_ref, emb_ref, out_ref):
    at = at_ref[...]
    ioc = jax.lax.broadcasted_iota(jnp.int32, (RB, 32), 1)
    oh = (at == ioc).astype(jnp.float32)
    out_ref[...] = _dot(oh, emb_ref[...])


def _geometry_kernel(nrbf, ps_ref, pr_ref, vm_ref, geo_ref):
    ps = ps_ref[...]
    pr = pr_ref[...]
    vec = ps[:, 0:3] - pr[:, 0:3]
    d2 = jnp.sum(vec * vec, axis=1, keepdims=True)
    dist = jnp.sqrt(d2 + 1e-12)
    dirv = vec / dist
    step = CUTOFF / (nrbf - 1)
    centers = jax.lax.broadcasted_iota(
        jnp.int32, (1, 24), 1).astype(jnp.float32) * step
    rbf = jnp.exp(-10.0 * (dist - centers) ** 2)
    fcut = 0.5 * (jnp.cos(jnp.pi * jnp.clip(dist / CUTOFF, 0.0, 1.0)) + 1.0)
    vm = vm_ref[...]
    geo_ref[:, 0:nrbf] = (rbf * fcut * vm)[:, 0:nrbf]
    geo_ref[:, nrbf:nrbf + 1] = fcut * vm
    geo_ref[:, nrbf + 1:24] = dirv * vm


def _edge_kernel(nrbf, h, has_xv, *refs):
    if has_xv:
        geo_ref, ps_ref, xv_ref, wf_ref, dxs_ref, dxv_ref = refs
    else:
        geo_ref, ps_ref, wf_ref, dxs_ref, dxv_ref = refs
    geo = geo_ref[...]
    wr = _dot(geo, wf_ref[...])
    m = ps_ref[...] * wr
    ds = m[:, 0:h]
    dvv = m[:, h:2 * h]
    dvs = m[:, 2 * h:3 * h]
    dx = geo[:, nrbf + 1:nrbf + 2]
    dy = geo[:, nrbf + 2:nrbf + 3]
    dz = geo[:, nrbf + 3:nrbf + 4]
    if has_xv:
        xv = xv_ref[...]
        dvx = dvv * dx + dvs * xv[:, 0:h]
        dvy = dvv * dy + dvs * xv[:, h:2 * h]
        dvz = dvv * dz + dvs * xv[:, 2 * h:3 * h]
    else:
        dvx = dvv * dx
        dvy = dvv * dy
        dvz = dvv * dz

    def red(a):
        return jnp.sum(a.reshape(BN, MAXNB, h), axis=1)

    dxs_ref[...] = red(ds)
    dxv_ref[:, 0:h] = red(dvx)
    dxv_ref[:, h:2 * h] = red(dvy)
    dxv_ref[:, 2 * h:3 * h] = red(dvz)


def _phi_kernel(xs_ref, w1_ref, b1_ref, w2_ref, b2_ref, out_ref):
    hid = _silu(_dot(xs_ref[...], w1_ref[...]) + b1_ref[...])
    out_ref[...] = _dot(hid, w2_ref[...]) + b2_ref[...]


def _update_kernel(h, xs_ref, xv_ref, dxs_ref, dxv_ref, wu_ref, wv_ref,
                   w1a_ref, w1b_ref, b1_ref, w2_ref, b2_ref,
                   xso_ref, xvo_ref):
    xs1 = xs_ref[...] + dxs_ref[...]
    xv1 = xv_ref[...] + dxv_ref[...]
    wu = wu_ref[...]
    wv = wv_ref[...]
    xvx = xv1[:, 0:h]
    xvy = xv1[:, h:2 * h]
    xvz = xv1[:, 2 * h:3 * h]
    ux = _dot(xvx, wu)
    uy = _dot(xvy, wu)
    uz = _dot(xvz, wu)
    vx = _dot(xvx, wv)
    vy = _dot(xvy, wv)
    vz = _dot(xvz, wv)
    vn = jnp.sqrt(vx * vx + vy * vy + vz * vz + 1e-12)
    hid = _silu(_dot(xs1, w1a_ref[...]) + _dot(vn, w1b_ref[...]) + b1_ref[...])
    a = _dot(hid, w2_ref[...]) + b2_ref[...]
    a_ss = a[:, 0:h]
    a_sv = a[:, h:2 * h]
    a_vv = a[:, 2 * h:3 * h]
    uv = ux * vx + uy * vy + uz * vz
    xso_ref[...] = xs1 + a_ss + a_sv * uv
    xvo_ref[:, 0:h] = xvx + a_vv * ux
    xvo_ref[:, h:2 * h] = xvy + a_vv * uy
    xvo_ref[:, 2 * h:3 * h] = xvz + a_vv * uz


def _head_kernel(xs_ref, at_ref, ft_ref, w1_ref, b1_ref, w2_ref, b2_ref,
                 sp_ref, out_ref):
    pid = pl.program_id(0)
    hid = _silu(_dot(xs_ref[...], w1_ref[...]) + b1_ref[...])
    e = (_dot(hid, w2_ref[...]) + b2_ref[...]) * ft_ref[...]
    ioc = jax.lax.broadcasted_iota(jnp.int32, (RB, 32), 1)
    oh = (at_ref[...] == ioc).astype(jnp.float32)
    ae = _dot(oh, sp_ref[...])
    s = jnp.sum(e + ae)

    @pl.when(pid == 0)
    def _():
        out_ref[...] = jnp.zeros_like(out_ref)

    out_ref[...] += s


def kernel(at_no, pos, shifts, batch, at_filter, params):
    n = pos.shape[0]
    h = params["embed"].shape[1]
    z = params["atom_sp"].shape[0]
    nrbf = params["msg"][0]["Wf"].shape[0]
    nlayers = len(params["msg"])

    np_ = ((n + RB - 1) // RB) * RB
    e = n * MAXNB
    ep = np_ * MAXNB
    f32 = jnp.float32

    src, valid = _build_graph(pos, CUTOFF, batch, MAXNB)
    src2d = jnp.concatenate([src, jnp.zeros((ep - e,), jnp.int32)])
    valid_f = jnp.zeros((ep, 1), f32).at[:e, 0].set(valid.astype(f32))

    pos_t = jnp.zeros((np_, 128), f32).at[:n, 0:3].set(pos + shifts)
    pos_d = jnp.zeros((np_, 4), f32).at[:n, 0:3].set(pos)
    pos_rep = jnp.repeat(pos_d, MAXNB, axis=0)

    at_p = jnp.full((np_, 1), z, jnp.int32).at[:n, 0].set(at_no)
    ft_p = jnp.zeros((np_, 1), f32).at[:n, 0].set(at_filter.astype(f32))
    emb_p = jnp.zeros((32, h), f32).at[:z].set(params["embed"])
    sp_p = jnp.zeros((32, 1), f32).at[:z, 0].set(params["atom_sp"])

    ngrid = np_ // RB
    egrid = ep // (BN * MAXNB)
    be = BN * MAXNB

    def full(shape):
        return pl.BlockSpec(shape, lambda i: tuple(0 for _ in shape))

    def rows(shape):
        return pl.BlockSpec(shape, lambda i: (i,) + tuple(0 for _ in shape[1:]))

    xs = pl.pallas_call(
        _embed_kernel,
        grid=(ngrid,),
        in_specs=[rows((RB, 1)), full((32, h))],
        out_specs=rows((RB, h)),
        out_shape=jax.ShapeDtypeStruct((np_, h), f32),
    )(at_p, emb_p)

    ps_g = _gather_rows(pos_t, src2d)
    geo = pl.pallas_call(
        functools.partial(_geometry_kernel, nrbf),
        grid=(egrid,),
        in_specs=[rows((be, 128)), rows((be, 4)), rows((be, 1))],
        out_specs=rows((be, 24)),
        out_shape=jax.ShapeDtypeStruct((ep, 24), f32),
    )(ps_g, pos_rep, valid_f)

    xv = jnp.zeros((np_, 3 * h), f32)
    for l in range(nlayers):
        p = params["msg"][l]
        wfp = (jnp.zeros((24, 3 * h), f32)
               .at[:nrbf].set(p["Wf"]).at[nrbf].set(p["bf"]))
        phi = pl.pallas_call(
            _phi_kernel,
            grid=(ngrid,),
            in_specs=[rows((RB, h)), full((h, h)), full((1, h)),
                      full((h, 3 * h)), full((1, 3 * h))],
            out_specs=rows((RB, 3 * h)),
            out_shape=jax.ShapeDtypeStruct((np_, 3 * h), f32),
        )(xs, p["W1"], p["b1"].reshape(1, h), p["W2"],
          p["b2"].reshape(1, 3 * h))

        phi_src = _gather_rows(phi, src2d)
        if l == 0:
            ins = [geo, phi_src, wfp]
            specs = [rows((be, 24)), rows((be, 3 * h)), full((24, 3 * h))]
        else:
            xv_src = _gather_rows(xv, src2d)
            ins = [geo, phi_src, xv_src, wfp]
            specs = [rows((be, 24)), rows((be, 3 * h)), rows((be, 3 * h)),
                     full((24, 3 * h))]
        dxs, dxv = pl.pallas_call(
            functools.partial(_edge_kernel, nrbf, h, l > 0),
            grid=(egrid,),
            in_specs=specs,
            out_specs=[rows((BN, h)), rows((BN, 3 * h))],
            out_shape=[jax.ShapeDtypeStruct((np_, h), f32),
                       jax.ShapeDtypeStruct((np_, 3 * h), f32)],
        )(*ins)

        q = params["upd"][l]
        xs, xv = pl.pallas_call(
            functools.partial(_update_kernel, h),
            grid=(ngrid,),
            in_specs=[rows((RB, h)), rows((RB, 3 * h)), rows((RB, h)),
                      rows((RB, 3 * h)), full((h, h)), full((h, h)),
                      full((h, h)), full((h, h)), full((1, h)),
                      full((h, 3 * h)), full((1, 3 * h))],
            out_specs=[rows((RB, h)), rows((RB, 3 * h))],
            out_shape=[jax.ShapeDtypeStruct((np_, h), f32),
                       jax.ShapeDtypeStruct((np_, 3 * h), f32)],
        )(xs, xv, dxs, dxv, q["WU"], q["WV"], q["W1"][:h], q["W1"][h:],
          q["b1"].reshape(1, h), q["W2"], q["b2"].reshape(1, 3 * h))

    o = params["out"]
    h2 = o["W1"].shape[1]
    out = pl.pallas_call(
        _head_kernel,
        grid=(ngrid,),
        in_specs=[rows((RB, h)), rows((RB, 1)), rows((RB, 1)),
                  full((h, h2)), full((1, h2)), full((h2, 1)), full((1, 1)),
                  full((32, 1))],
        out_specs=full((1, 1)),
        out_shape=jax.ShapeDtypeStruct((1, 1), f32),
    )(xs, at_p, ft_p, o["W1"], o["b1"].reshape(1, h2), o["W2"],
      o["b2"].reshape(1, 1), sp_p)
    return out.reshape(1)
